# baseline (speedup 1.0000x reference)
import functools

import jax
import jax.numpy as jnp
from jax import lax
from jax.experimental import pallas as pl
from jax.experimental.pallas import tpu as pltpu
from jax.experimental.pallas import tpu_sc as plsc

VOCAB = 400002
EMBED = 50
EMBED_P = 56
BATCH = 4096
SEQ = 200

NC, NS = 2, 16
NW = NC * NS
NTOT = BATCH * SEQ
IDX_W = 128
GPC = 8
CHUNK = GPC * IDX_W
PER_W = NTOT // NW
N_CHUNKS = PER_W // CHUNK
IDX_ROWS_PER_W = PER_W // IDX_W

_mesh = plsc.VectorSubcoreMesh(
    core_axis_name="c", subcore_axis_name="s", num_cores=NC, num_subcores=NS
)


@functools.partial(
    pl.kernel,
    out_type=jax.ShapeDtypeStruct((NTOT, EMBED_P), jnp.float32),
    mesh=_mesh,
    scratch_types=[
        pltpu.VMEM((GPC, IDX_W), jnp.int32),
        pltpu.VMEM((CHUNK, EMBED_P), jnp.float32),
        pltpu.SemaphoreType.DMA,
    ],
    compiler_params=pltpu.CompilerParams(use_tc_tiling_on_sc=False),
)
def _gather_kernel(table_hbm, idx_hbm, out_hbm, idx_v, rows_v, sem):
    wid = lax.axis_index("s") * NC + lax.axis_index("c")
    irow0 = wid * IDX_ROWS_PER_W
    orow0 = wid * PER_W

    @pl.loop(0, N_CHUNKS)
    def _chunk(m):
        pltpu.sync_copy(idx_hbm.at[pl.ds(irow0 + m * GPC, GPC)], idx_v)
        copies = [
            pltpu.async_copy(
                table_hbm.at[idx_v.at[g]],
                rows_v.at[pl.ds(g * IDX_W, IDX_W)],
                sem,
            )
            for g in range(GPC)
        ]
        for c in copies:
            c.wait()
        pltpu.sync_copy(rows_v, out_hbm.at[pl.ds(orow0 + m * CHUNK, CHUNK)])


def kernel(input_ids, word_embeddings):
    idx2d = input_ids.reshape(NTOT // IDX_W, IDX_W).astype(jnp.int32)
    table_p = jnp.pad(word_embeddings, ((0, 0), (0, EMBED_P - EMBED)))
    out = _gather_kernel(table_p, idx2d)
    return out[:, :EMBED].reshape(BATCH, SEQ, EMBED)
